# baseline (speedup 1.0000x reference)
import functools

import jax
import jax.numpy as jnp
from jax import lax
from jax.experimental import pallas as pl
from jax.experimental.pallas import tpu as pltpu
from jax.experimental.pallas import tpu_sc as plsc

N = 10000
E = 160000
NG = 16
DP = 512
D2P = 1024
D4P = 2048
NDEG = 10240

BM = 1000
MB = N // BM

NCORE = 2
NSUB = 16
EPT = E // NSUB
NBR = 80
KB = 64
NBLK = 160
NP = 10240
ROWS_PT = NP // NSUB
EPW = E // (NCORE * NSUB)
ACC2 = 2 * NDEG + 512

_f32 = jnp.float32
_i32 = jnp.int32


def _degree_call(dsts64):
    mesh = plsc.VectorSubcoreMesh(core_axis_name="c", subcore_axis_name="s")

    @functools.partial(
        pl.kernel,
        out_type=jax.ShapeDtypeStruct((NCORE * 2 * NDEG,), _f32),
        mesh=mesh,
        scratch_types=[
            pltpu.VMEM((5120,), _i32),
            pltpu.VMEM((40, 128), _i32),
            pltpu.VMEM((128,), _f32),
            pltpu.VMEM((1312,), _f32),
            pltpu.VMEM_SHARED((ACC2,), _f32),
        ],
    )
    def k(d_hbm, out_hbm, raw_v, didx, ones_v, zv, acc2):
        cid = lax.axis_index("c")
        sid = lax.axis_index("s")
        wid = cid * NSUB + sid

        @pl.loop(0, 8)
        def _fill_ones(i):
            ones_v[pl.ds(16 * i, 16)] = jnp.ones((16,), _f32)

        @pl.loop(0, 82)
        def _fill_zero(i):
            zv[pl.ds(16 * i, 16)] = jnp.zeros((16,), _f32)

        pltpu.sync_copy(zv, acc2.at[pl.ds(sid * 1312, 1312)])
        plsc.subcore_barrier()

        for es in range(2):
            r = es * (NCORE * NSUB) + wid
            pltpu.sync_copy(d_hbm.at[r], raw_v)

            @pl.loop(0, 40)
            def _blk(bk):
                for j in range(8):
                    pos0 = bk * 128 + 16 * j
                    pos = pos0 + lax.iota(_i32, 16)
                    v = raw_v[pl.ds(pos0, 16)]
                    adj = jnp.where(pos < EPW, v + es * NDEG,
                                    2 * NDEG + 16 * j + lax.iota(_i32, 16))
                    didx[bk, pl.ds(16 * j, 16)] = adj
                pltpu.sync_copy(ones_v, acc2.at[didx.at[bk]], add=True)

        plsc.subcore_barrier()
        pltpu.sync_copy(acc2.at[pl.ds(sid * 1280, 1280)],
                        out_hbm.at[pl.ds(cid * 2 * NDEG + sid * 1280, 1280)])

    return k(dsts64)


def _msgpass_call(h_flat, src3, dst3, nck):
    half = nck // 2
    mesh = plsc.VectorSubcoreMesh(core_axis_name="c", subcore_axis_name="s")

    @functools.partial(
        pl.kernel,
        out_type=jax.ShapeDtypeStruct((nck * NP, 128), _f32),
        mesh=mesh,
        scratch_types=[
            pltpu.VMEM((NBR, 128), _i32),
            pltpu.VMEM((NBR, 128), _i32),
            pltpu.VMEM((3, KB), _i32),
            pltpu.VMEM((3, KB), _i32),
            pltpu.VMEM((3, KB, 128), _f32),
            pltpu.VMEM_SHARED((NP, 128), _f32),
            pltpu.SemaphoreType.DMA,
            pltpu.SemaphoreType.DMA,
        ],
    )
    def k(h_hbm, src_hbm, dst_hbm, out_hbm,
          src_v, dst_raw, idxb, didx, buf, acc, gsem, ssem):
        cid = lax.axis_index("c")
        sid = lax.axis_index("s")
        pltpu.sync_copy(src_hbm.at[sid], src_v)
        pltpu.sync_copy(dst_hbm.at[sid], dst_raw)

        base = sid * ROWS_PT

        for jj in range(half):
            c = cid * half + jj

            @pl.loop(0, KB)
            def _zz(i):
                for j in range(8):
                    buf[0, i, pl.ds(16 * j, 16)] = jnp.zeros((16,), _f32)
            for t in range(ROWS_PT // KB):
                pltpu.sync_copy(buf.at[0], acc.at[pl.ds(base + KB * t, KB)])
            plsc.subcore_barrier()

            coff = c * N

            def _prep(b, slot, coff=coff):
                r = lax.div(b, 2)
                off = lax.rem(b, 2) * KB
                for j in range(KB // 16):
                    v = src_v[r, pl.ds(off + 16 * j, 16)]
                    idxb[slot, pl.ds(16 * j, 16)] = v + coff
                    d = dst_raw[r, pl.ds(off + 16 * j, 16)]
                    didx[slot, pl.ds(16 * j, 16)] = d

            def _wait_g(slot):
                pltpu.make_async_copy(h_hbm.at[idxb.at[slot]], buf.at[slot],
                                      gsem).wait()

            def _wait_s(slot):
                pltpu.make_async_copy(buf.at[slot], acc.at[didx.at[slot]],
                                      ssem).wait()

            _prep(0, 0)
            pltpu.async_copy(h_hbm.at[idxb.at[0]], buf.at[0], gsem)

            @pl.loop(0, NBLK)
            def _pipe(b):
                sb = lax.rem(b, 3)
                _wait_g(sb)

                @pl.when(b >= 2)
                def _():
                    _wait_s(lax.rem(b + 1, 3))
                nb = b + 1

                @pl.when(nb < NBLK)
                def _():
                    ns = lax.rem(nb, 3)
                    _prep(nb, ns)
                    pltpu.async_copy(h_hbm.at[idxb.at[ns]], buf.at[ns], gsem)

                pltpu.async_copy(buf.at[sb], acc.at[didx.at[sb]], ssem,
                                 add=True)

            _wait_s(lax.rem(NBLK - 2, 3))
            _wait_s(lax.rem(NBLK - 1, 3))
            plsc.subcore_barrier()
            pltpu.sync_copy(acc.at[pl.ds(base, ROWS_PT)],
                            out_hbm.at[pl.ds(c * NP + base, ROWS_PT)])
            plsc.subcore_barrier()

    return k(h_flat, src3, dst3)


def _dinv_call(degparts):
    def body(d_ref, o_ref):
        for es in range(2):
            o_ref[es] = lax.rsqrt(d_ref[es] + d_ref[2 + es] + 1.0)

    return pl.pallas_call(
        body, out_shape=jax.ShapeDtypeStruct((2, 80, 128), _f32))(degparts)


def _encoder_call(x, wbig, bfeat):
    KE = 51

    def body(x_ref, w_ref, b_ref, o_ref):
        kk = pl.program_id(1)
        xb = x_ref[...]
        lane = lax.broadcasted_iota(_i32, (BM, 128), 1)
        xb = jnp.where(kk * 128 + lane < 6485, xb, 0.0)
        prod = jnp.dot(xb, w_ref[...], preferred_element_type=_f32)

        @pl.when(kk == 0)
        def _():
            o_ref[...] = prod

        @pl.when(kk > 0)
        def _():
            o_ref[...] = o_ref[...] + prod

        @pl.when(kk == KE - 1)
        def _():
            o_ref[...] = jnp.maximum(o_ref[...] + b_ref[...], 0.0)

    return pl.pallas_call(
        body,
        grid=(MB, KE),
        in_specs=[
            pl.BlockSpec((BM, 128), lambda m, k: (m, k)),
            pl.BlockSpec((128, DP), lambda m, k: (k, 0)),
            pl.BlockSpec((1, DP), lambda m, k: (0, 0)),
        ],
        out_specs=pl.BlockSpec((BM, DP), lambda m, k: (m, 0)),
        out_shape=jax.ShapeDtypeStruct((N, DP), _f32),
        compiler_params=pltpu.CompilerParams(
            dimension_semantics=("parallel", "arbitrary")),
    )(x, wbig, bfeat)


def _producer_call(x, w, dinv_out, ko):
    ki = x.shape[1] // 128

    def body(x_ref, w_ref, dv_ref, o_ref, acc):
        kk = pl.program_id(1)
        prod = jnp.dot(x_ref[...], w_ref[...], preferred_element_type=_f32)

        @pl.when(kk == 0)
        def _():
            acc[...] = prod

        @pl.when(kk > 0)
        def _():
            acc[...] = acc[...] + prod

        @pl.when(kk == ki - 1)
        def _():
            s = dv_ref[...] * acc[...]
            for cc in range(ko):
                o_ref[cc] = s[:, 128 * cc:128 * (cc + 1)]

    return pl.pallas_call(
        body,
        grid=(MB, ki),
        in_specs=[
            pl.BlockSpec((BM, 128), lambda m, k: (m, k)),
            pl.BlockSpec((128, 128 * ko), lambda m, k: (k, 0)),
            pl.BlockSpec((BM, 1), lambda m, k: (m, 0)),
        ],
        out_specs=pl.BlockSpec((ko, BM, 128), lambda m, k: (0, m, 0)),
        out_shape=jax.ShapeDtypeStruct((ko, N, 128), _f32),
        scratch_shapes=[pltpu.VMEM((BM, 128 * ko), _f32)],
        compiler_params=pltpu.CompilerParams(
            dimension_semantics=("parallel", "arbitrary")),
    )(x, w, dinv_out)


def _conprod_call(a, h, dvin, bias, w, dinv_out, ko):
    ki = a.shape[0]

    def body(a_ref, h_ref, di_ref, b_ref, w_ref, do_ref, o_ref, acc):
        kk = pl.program_id(1)
        z = di_ref[0] * (a_ref[0] + h_ref[0]) + b_ref[0]
        z = jnp.maximum(z, 0.0)
        prod = jnp.dot(z, w_ref[...], preferred_element_type=_f32)

        @pl.when(kk == 0)
        def _():
            acc[...] = prod

        @pl.when(kk > 0)
        def _():
            acc[...] = acc[...] + prod

        @pl.when(kk == ki - 1)
        def _():
            s = do_ref[...] * acc[...]
            for cc in range(ko):
                o_ref[cc] = s[:, 128 * cc:128 * (cc + 1)]

    return pl.pallas_call(
        body,
        grid=(MB, ki),
        in_specs=[
            pl.BlockSpec((1, BM, 128), lambda m, k: (k, m, 0)),
            pl.BlockSpec((1, BM, 128), lambda m, k: (k, m, 0)),
            pl.BlockSpec((1, BM, 1), lambda m, k: (k, m, 0)),
            pl.BlockSpec((1, 1, 128), lambda m, k: (k, 0, 0)),
            pl.BlockSpec((128, 128 * ko), lambda m, k: (k, 0)),
            pl.BlockSpec((BM, 1), lambda m, k: (m, 0)),
        ],
        out_specs=pl.BlockSpec((ko, BM, 128), lambda m, k: (0, m, 0)),
        out_shape=jax.ShapeDtypeStruct((ko, N, 128), _f32),
        scratch_shapes=[pltpu.VMEM((BM, 128 * ko), _f32)],
        compiler_params=pltpu.CompilerParams(
            dimension_semantics=("parallel", "arbitrary")),
    )(a, h, dvin, bias, w, dinv_out)


def _poolhead_call(a5, h5, dv1, bc3c, batch2d, wf1, bf1, gam, bet, wf2, bf2):
    def body(a_ref, h_ref, dv_ref, bc_ref, bt_ref, w1_ref, b1_ref, g_ref,
             be_ref, w2_ref, b2_ref, o_ref, accs, accc):
        m = pl.program_id(0)
        dvb = dv_ref[...]
        parts = []
        for cc in range(16):
            zc = jnp.maximum(dvb * (a_ref[cc] + h_ref[cc]) + bc_ref[cc], 0.0)
            parts.append(zc)
        z = jnp.concatenate(parts, axis=1)
        gid = lax.broadcasted_iota(_i32, (NG, BM), 0)
        p = (gid == bt_ref[0]).astype(_f32)
        ps = jnp.dot(p, z, preferred_element_type=_f32,
                     precision=lax.Precision.HIGHEST)
        pc = jnp.sum(p, axis=1, keepdims=True)

        @pl.when(m == 0)
        def _():
            accs[...] = ps
            accc[...] = pc

        @pl.when(m > 0)
        def _():
            accs[...] = accs[...] + ps
            accc[...] = accc[...] + pc

        @pl.when(m == MB - 1)
        def _():
            zp = accs[...] / jnp.maximum(accc[...], 1.0)
            hh = jnp.dot(zp, w1_ref[...], preferred_element_type=_f32)
            hh = hh + b1_ref[...]
            mu = jnp.mean(hh, axis=0, keepdims=True)
            var = jnp.mean((hh - mu) ** 2, axis=0, keepdims=True)
            hn = (hh - mu) * lax.rsqrt(var + 1e-5) * g_ref[...] + be_ref[...]
            hn = jnp.maximum(hn, 0.0)
            oo = jnp.dot(hn, w2_ref[...], preferred_element_type=_f32)
            oo = oo + b2_ref[...]
            o_ref[...] = 1.0 / (1.0 + jnp.exp(-oo))

    return pl.pallas_call(
        body,
        grid=(MB,),
        in_specs=[
            pl.BlockSpec((16, BM, 128), lambda m: (0, m, 0)),
            pl.BlockSpec((16, BM, 128), lambda m: (0, m, 0)),
            pl.BlockSpec((BM, 1), lambda m: (m, 0)),
            pl.BlockSpec((16, 128), lambda m: (0, 0)),
            pl.BlockSpec((1, 1, BM), lambda m: (m, 0, 0)),
            pl.BlockSpec((D4P, 1024), lambda m: (0, 0)),
            pl.BlockSpec((1, 1024), lambda m: (0, 0)),
            pl.BlockSpec((1, 1024), lambda m: (0, 0)),
            pl.BlockSpec((1, 1024), lambda m: (0, 0)),
            pl.BlockSpec((1024, 512), lambda m: (0, 0)),
            pl.BlockSpec((1, 512), lambda m: (0, 0)),
        ],
        out_specs=pl.BlockSpec((NG, 512), lambda m: (0, 0)),
        out_shape=jax.ShapeDtypeStruct((NG, 512), _f32),
        scratch_shapes=[pltpu.VMEM((NG, D4P), _f32),
                        pltpu.VMEM((NG, 1), _f32)],
    )(a5, h5, dv1, bc3c, batch2d, wf1, bf1, gam, bet, wf2, bf2)


def kernel(prot_x, prot_edge_index, edge_index_replace, prot_batch,
           W1, b1, W2, b2, W3, b3, Wc1, bc1, Wc2, bc2, Wa1, ba1, Wa2, ba2,
           Wc3, bc3, Wf1, bf1, gamma, beta, Wf2, bf2):
    z = jnp.zeros
    wbig = z((6528, DP), _f32)
    wbig = wbig.at[0:21, 0:21].set(W2)
    wbig = wbig.at[21:6165, 21:149].set(W1)
    wbig = wbig.at[6165:6485, 149:469].set(W3)
    bfeat = z((DP,), _f32).at[0:21].set(b2).at[21:149].set(b1)
    bfeat = bfeat.at[149:469].set(b3).reshape(1, DP)

    wc1p = z((DP, DP), _f32).at[:469, :469].set(Wc1)
    wa1p = z((DP, DP), _f32).at[:469, :469].set(Wa1)
    wc2p = z((DP, D2P), _f32).at[:469, :938].set(Wc2)
    wa2p = z((DP, D2P), _f32).at[:469, :938].set(Wa2)
    wc3p = z((D4P, D4P), _f32)
    wc3p = wc3p.at[0:938, 0:1876].set(Wc3[0:938])
    wc3p = wc3p.at[1024:1962, 0:1876].set(Wc3[938:1876])

    bc1c = z((DP,), _f32).at[:469].set(bc1).reshape(4, 1, 128)
    ba1c = z((DP,), _f32).at[:469].set(ba1).reshape(4, 1, 128)
    bc2c = z((D2P,), _f32).at[:938].set(bc2).reshape(8, 1, 128)
    ba2c = z((D2P,), _f32).at[:938].set(ba2).reshape(8, 1, 128)
    bc3c = z((D4P,), _f32).at[:1876].set(bc3).reshape(16, 128)

    wf1p = z((D4P, 1024), _f32).at[:1876].set(Wf1)
    wf2p = z((1024, 512), _f32).at[:, :486].set(Wf2)
    bf2p = z((512,), _f32).at[:486].set(bf2).reshape(1, 512)
    bf1r = bf1.reshape(1, 1024)
    gam = gamma.reshape(1, 1024)
    bet = beta.reshape(1, 1024)

    spad = jnp.zeros((NSUB, NBLK * KB - EPT), _i32)
    dpad = jnp.broadcast_to(N + jnp.arange(NBLK * KB - EPT, dtype=_i32),
                            (NSUB, NBLK * KB - EPT))

    def _pe(e):
        return jnp.concatenate(
            [e[0].reshape(NSUB, EPT), spad], axis=1).reshape(NSUB, NBR, 128),                jnp.concatenate(
            [e[1].reshape(NSUB, EPT), dpad], axis=1).reshape(NSUB, NBR, 128)

    src1, dst1 = _pe(prot_edge_index)
    src2, dst2 = _pe(edge_index_replace)
    dsts64 = jnp.pad(jnp.concatenate(
        [prot_edge_index[1], edge_index_replace[1]]).reshape(
            2 * NCORE * NSUB, EPW), ((0, 0), (0, 120)))

    degparts = _degree_call(dsts64).reshape(4, 80, 128)
    dv = _dinv_call(degparts)
    dinv1 = dv[0].reshape(NDEG)[:N].reshape(N, 1)
    dinv2 = dv[1].reshape(NDEG)[:N].reshape(N, 1)

    feat = _encoder_call(prot_x, wbig, bfeat)

    h1 = _producer_call(feat, wc1p, dinv1, 4)
    h3 = _producer_call(feat, wa1p, dinv2, 4)
    a1 = _msgpass_call(h1.reshape(4 * N, 128), src1, dst1, 4).reshape(4, NP, 128)
    a3 = _msgpass_call(h3.reshape(4 * N, 128), src2, dst2, 4).reshape(4, NP, 128)

    dv1s4 = jnp.broadcast_to(dinv1[None], (4, N, 1))
    dv2s4 = jnp.broadcast_to(dinv2[None], (4, N, 1))
    h2 = _conprod_call(a1, h1, dv1s4, bc1c, wc2p, dinv1, 8)
    h4 = _conprod_call(a3, h3, dv2s4, ba1c, wa2p, dinv2, 8)
    a2 = _msgpass_call(h2.reshape(8 * N, 128), src1, dst1, 8).reshape(8, NP, 128)
    a4 = _msgpass_call(h4.reshape(8 * N, 128), src2, dst2, 8).reshape(8, NP, 128)

    a24 = jnp.concatenate([a2, a4], axis=0)
    h24 = jnp.concatenate([h2, h4], axis=0)
    dv24 = jnp.concatenate([jnp.broadcast_to(dinv1[None], (8, N, 1)),
                            jnp.broadcast_to(dinv2[None], (8, N, 1))], axis=0)
    b24 = jnp.concatenate([bc2c, ba2c], axis=0)
    h5 = _conprod_call(a24, h24, dv24, b24, wc3p, dinv1, 16)
    a5 = _msgpass_call(h5.reshape(16 * N, 128), src1, dst1, 16).reshape(
        16, NP, 128)

    out = _poolhead_call(a5, h5, dinv1, bc3c, prot_batch.reshape(MB, 1, BM),
                         wf1p, bf1r, gam, bet, wf2p, bf2p)
    return out[:, :486]


# speedup vs baseline: 1.1202x; 1.1202x over previous
import functools

import jax
import jax.numpy as jnp
from jax import lax
from jax.experimental import pallas as pl
from jax.experimental.pallas import tpu as pltpu
from jax.experimental.pallas import tpu_sc as plsc

N = 10000
E = 160000
NG = 16
DP = 512
D2P = 1024
D4P = 2048
NDEG = 10240

BM = 1000
MB = N // BM

NCORE = 2
NSUB = 16
EPT = E // NSUB
NBR = 80
KB = 128
NBLK = 80
NP = 10240
ROWS_PT = NP // NSUB
EPW = E // (NCORE * NSUB)
ACC2 = 2 * NDEG + 512

_f32 = jnp.float32
_i32 = jnp.int32


def _degree_call(dsts64):
    mesh = plsc.VectorSubcoreMesh(core_axis_name="c", subcore_axis_name="s")

    @functools.partial(
        pl.kernel,
        out_type=jax.ShapeDtypeStruct((NCORE * 2 * NDEG,), _f32),
        mesh=mesh,
        scratch_types=[
            pltpu.VMEM((5120,), _i32),
            pltpu.VMEM((40, 128), _i32),
            pltpu.VMEM((128,), _f32),
            pltpu.VMEM((1312,), _f32),
            pltpu.VMEM_SHARED((ACC2,), _f32),
        ],
    )
    def k(d_hbm, out_hbm, raw_v, didx, ones_v, zv, acc2):
        cid = lax.axis_index("c")
        sid = lax.axis_index("s")
        wid = cid * NSUB + sid

        @pl.loop(0, 8)
        def _fill_ones(i):
            ones_v[pl.ds(16 * i, 16)] = jnp.ones((16,), _f32)

        @pl.loop(0, 82)
        def _fill_zero(i):
            zv[pl.ds(16 * i, 16)] = jnp.zeros((16,), _f32)

        pltpu.sync_copy(zv, acc2.at[pl.ds(sid * 1312, 1312)])
        plsc.subcore_barrier()

        for es in range(2):
            r = es * (NCORE * NSUB) + wid
            pltpu.sync_copy(d_hbm.at[r], raw_v)

            @pl.loop(0, 40)
            def _blk(bk):
                for j in range(8):
                    pos0 = bk * 128 + 16 * j
                    pos = pos0 + lax.iota(_i32, 16)
                    v = raw_v[pl.ds(pos0, 16)]
                    adj = jnp.where(pos < EPW, v + es * NDEG,
                                    2 * NDEG + 16 * j + lax.iota(_i32, 16))
                    didx[bk, pl.ds(16 * j, 16)] = adj
                pltpu.sync_copy(ones_v, acc2.at[didx.at[bk]], add=True)

        plsc.subcore_barrier()
        pltpu.sync_copy(acc2.at[pl.ds(sid * 1280, 1280)],
                        out_hbm.at[pl.ds(cid * 2 * NDEG + sid * 1280, 1280)])

    return k(dsts64)


def _msgpass_call(h_flat, ed3, nck):
    half = nck // 2
    mesh = plsc.VectorSubcoreMesh(core_axis_name="c", subcore_axis_name="s")

    @functools.partial(
        pl.kernel,
        out_type=jax.ShapeDtypeStruct((nck * NP, 128), _f32),
        mesh=mesh,
        scratch_types=[
            pltpu.VMEM((NBR, 128), _i32),
            pltpu.VMEM((2, KB), _i32),
            pltpu.VMEM((2, KB), _i32),
            pltpu.VMEM((2, KB, 128), _f32),
            pltpu.VMEM_SHARED((NP, 128), _f32),
            pltpu.SemaphoreType.DMA,
            pltpu.SemaphoreType.DMA,
        ],
    )
    def k(h_hbm, ed_hbm, out_hbm,
          ed_v, idxb, didx, buf, acc, gsem, ssem):
        cid = lax.axis_index("c")
        sid = lax.axis_index("s")
        pltpu.sync_copy(ed_hbm.at[sid], ed_v)

        base = sid * ROWS_PT

        for jj in range(half):
            c = cid * half + jj

            @pl.loop(0, KB)
            def _zz(i):
                for j in range(8):
                    buf[0, i, pl.ds(16 * j, 16)] = jnp.zeros((16,), _f32)
            for t in range(ROWS_PT // KB):
                pltpu.sync_copy(buf.at[0], acc.at[pl.ds(base + KB * t, KB)])
            plsc.subcore_barrier()

            coff = c * N

            def _prep(b, slot, coff=coff):
                for j in range(KB // 16):
                    v = ed_v[b, pl.ds(16 * j, 16)]
                    idxb[slot, pl.ds(16 * j, 16)] = (v & 0xFFFF) + coff
                    didx[slot, pl.ds(16 * j, 16)] = (
                        lax.shift_right_logical(v, 16))

            def _wait_g(slot):
                pltpu.make_async_copy(h_hbm.at[idxb.at[slot]], buf.at[slot],
                                      gsem).wait()

            def _wait_s(slot):
                pltpu.make_async_copy(buf.at[slot], acc.at[didx.at[slot]],
                                      ssem).wait()

            _prep(0, 0)
            pltpu.async_copy(h_hbm.at[idxb.at[0]], buf.at[0], gsem)

            @pl.loop(0, NBLK)
            def _pipe(b):
                sb = lax.rem(b, 2)
                _wait_g(sb)

                @pl.when(b >= 1)
                def _():
                    _wait_s(lax.rem(b + 1, 2))
                nb = b + 1

                @pl.when(nb < NBLK)
                def _():
                    ns = lax.rem(nb, 2)
                    _prep(nb, ns)
                    pltpu.async_copy(h_hbm.at[idxb.at[ns]], buf.at[ns], gsem)

                pltpu.async_copy(buf.at[sb], acc.at[didx.at[sb]], ssem,
                                 add=True)

            _wait_s(lax.rem(NBLK - 1, 2))
            plsc.subcore_barrier()
            pltpu.sync_copy(acc.at[pl.ds(base, ROWS_PT)],
                            out_hbm.at[pl.ds(c * NP + base, ROWS_PT)])
            plsc.subcore_barrier()

    return k(h_flat, ed3)


def _dinv_call(degparts):
    def body(d_ref, o_ref):
        for es in range(2):
            o_ref[es] = lax.rsqrt(d_ref[es] + d_ref[2 + es] + 1.0)

    return pl.pallas_call(
        body, out_shape=jax.ShapeDtypeStruct((2, 80, 128), _f32))(degparts)


def _encoder_call(x, wbig, bfeat):
    KE = 51

    def body(x_ref, w_ref, b_ref, o_ref):
        kk = pl.program_id(1)
        xb = x_ref[...]
        lane = lax.broadcasted_iota(_i32, (BM, 128), 1)
        xb = jnp.where(kk * 128 + lane < 6485, xb, 0.0)
        prod = jnp.dot(xb, w_ref[...], preferred_element_type=_f32)

        @pl.when(kk == 0)
        def _():
            o_ref[...] = prod

        @pl.when(kk > 0)
        def _():
            o_ref[...] = o_ref[...] + prod

        @pl.when(kk == KE - 1)
        def _():
            o_ref[...] = jnp.maximum(o_ref[...] + b_ref[...], 0.0)

    return pl.pallas_call(
        body,
        grid=(MB, KE),
        in_specs=[
            pl.BlockSpec((BM, 128), lambda m, k: (m, k)),
            pl.BlockSpec((128, DP), lambda m, k: (k, 0)),
            pl.BlockSpec((1, DP), lambda m, k: (0, 0)),
        ],
        out_specs=pl.BlockSpec((BM, DP), lambda m, k: (m, 0)),
        out_shape=jax.ShapeDtypeStruct((N, DP), _f32),
        compiler_params=pltpu.CompilerParams(
            dimension_semantics=("parallel", "arbitrary")),
    )(x, wbig, bfeat)


def _producer_call(x, w, dinv_out, ko):
    ki = x.shape[1] // 128

    def body(x_ref, w_ref, dv_ref, o_ref, acc):
        kk = pl.program_id(1)
        prod = jnp.dot(x_ref[...], w_ref[...], preferred_element_type=_f32)

        @pl.when(kk == 0)
        def _():
            acc[...] = prod

        @pl.when(kk > 0)
        def _():
            acc[...] = acc[...] + prod

        @pl.when(kk == ki - 1)
        def _():
            s = dv_ref[...] * acc[...]
            for cc in range(ko):
                o_ref[cc] = s[:, 128 * cc:128 * (cc + 1)]

    return pl.pallas_call(
        body,
        grid=(MB, ki),
        in_specs=[
            pl.BlockSpec((BM, 128), lambda m, k: (m, k)),
            pl.BlockSpec((128, 128 * ko), lambda m, k: (k, 0)),
            pl.BlockSpec((BM, 1), lambda m, k: (m, 0)),
        ],
        out_specs=pl.BlockSpec((ko, BM, 128), lambda m, k: (0, m, 0)),
        out_shape=jax.ShapeDtypeStruct((ko, N, 128), _f32),
        scratch_shapes=[pltpu.VMEM((BM, 128 * ko), _f32)],
        compiler_params=pltpu.CompilerParams(
            dimension_semantics=("parallel", "arbitrary")),
    )(x, w, dinv_out)


def _conprod_call(a, h, dvin, bias, w, dinv_out, ko):
    ki = a.shape[0]

    def body(a_ref, h_ref, di_ref, b_ref, w_ref, do_ref, o_ref, acc):
        kk = pl.program_id(1)
        z = di_ref[0] * (a_ref[0] + h_ref[0]) + b_ref[0]
        z = jnp.maximum(z, 0.0)
        prod = jnp.dot(z, w_ref[...], preferred_element_type=_f32)

        @pl.when(kk == 0)
        def _():
            acc[...] = prod

        @pl.when(kk > 0)
        def _():
            acc[...] = acc[...] + prod

        @pl.when(kk == ki - 1)
        def _():
            s = do_ref[...] * acc[...]
            for cc in range(ko):
                o_ref[cc] = s[:, 128 * cc:128 * (cc + 1)]

    return pl.pallas_call(
        body,
        grid=(MB, ki),
        in_specs=[
            pl.BlockSpec((1, BM, 128), lambda m, k: (k, m, 0)),
            pl.BlockSpec((1, BM, 128), lambda m, k: (k, m, 0)),
            pl.BlockSpec((1, BM, 1), lambda m, k: (k, m, 0)),
            pl.BlockSpec((1, 1, 128), lambda m, k: (k, 0, 0)),
            pl.BlockSpec((128, 128 * ko), lambda m, k: (k, 0)),
            pl.BlockSpec((BM, 1), lambda m, k: (m, 0)),
        ],
        out_specs=pl.BlockSpec((ko, BM, 128), lambda m, k: (0, m, 0)),
        out_shape=jax.ShapeDtypeStruct((ko, N, 128), _f32),
        scratch_shapes=[pltpu.VMEM((BM, 128 * ko), _f32)],
        compiler_params=pltpu.CompilerParams(
            dimension_semantics=("parallel", "arbitrary")),
    )(a, h, dvin, bias, w, dinv_out)


def _poolhead_call(a5, h5, dv1, bc3c, batch2d, wf1, bf1, gam, bet, wf2, bf2):
    def body(a_ref, h_ref, dv_ref, bc_ref, bt_ref, w1_ref, b1_ref, g_ref,
             be_ref, w2_ref, b2_ref, o_ref, accs, accc):
        m = pl.program_id(0)
        dvb = dv_ref[...]
        parts = []
        for cc in range(16):
            zc = jnp.maximum(dvb * (a_ref[cc] + h_ref[cc]) + bc_ref[cc], 0.0)
            parts.append(zc)
        z = jnp.concatenate(parts, axis=1)
        gid = lax.broadcasted_iota(_i32, (NG, BM), 0)
        p = (gid == bt_ref[0]).astype(_f32)
        ps = jnp.dot(p, z, preferred_element_type=_f32,
                     precision=lax.Precision.HIGHEST)
        pc = jnp.sum(p, axis=1, keepdims=True)

        @pl.when(m == 0)
        def _():
            accs[...] = ps
            accc[...] = pc

        @pl.when(m > 0)
        def _():
            accs[...] = accs[...] + ps
            accc[...] = accc[...] + pc

        @pl.when(m == MB - 1)
        def _():
            zp = accs[...] / jnp.maximum(accc[...], 1.0)
            hh = jnp.dot(zp, w1_ref[...], preferred_element_type=_f32)
            hh = hh + b1_ref[...]
            mu = jnp.mean(hh, axis=0, keepdims=True)
            var = jnp.mean((hh - mu) ** 2, axis=0, keepdims=True)
            hn = (hh - mu) * lax.rsqrt(var + 1e-5) * g_ref[...] + be_ref[...]
            hn = jnp.maximum(hn, 0.0)
            oo = jnp.dot(hn, w2_ref[...], preferred_element_type=_f32)
            oo = oo + b2_ref[...]
            o_ref[...] = 1.0 / (1.0 + jnp.exp(-oo))

    return pl.pallas_call(
        body,
        grid=(MB,),
        in_specs=[
            pl.BlockSpec((16, BM, 128), lambda m: (0, m, 0)),
            pl.BlockSpec((16, BM, 128), lambda m: (0, m, 0)),
            pl.BlockSpec((BM, 1), lambda m: (m, 0)),
            pl.BlockSpec((16, 128), lambda m: (0, 0)),
            pl.BlockSpec((1, 1, BM), lambda m: (m, 0, 0)),
            pl.BlockSpec((D4P, 1024), lambda m: (0, 0)),
            pl.BlockSpec((1, 1024), lambda m: (0, 0)),
            pl.BlockSpec((1, 1024), lambda m: (0, 0)),
            pl.BlockSpec((1, 1024), lambda m: (0, 0)),
            pl.BlockSpec((1024, 512), lambda m: (0, 0)),
            pl.BlockSpec((1, 512), lambda m: (0, 0)),
        ],
        out_specs=pl.BlockSpec((NG, 512), lambda m: (0, 0)),
        out_shape=jax.ShapeDtypeStruct((NG, 512), _f32),
        scratch_shapes=[pltpu.VMEM((NG, D4P), _f32),
                        pltpu.VMEM((NG, 1), _f32)],
    )(a5, h5, dv1, bc3c, batch2d, wf1, bf1, gam, bet, wf2, bf2)


def kernel(prot_x, prot_edge_index, edge_index_replace, prot_batch,
           W1, b1, W2, b2, W3, b3, Wc1, bc1, Wc2, bc2, Wa1, ba1, Wa2, ba2,
           Wc3, bc3, Wf1, bf1, gamma, beta, Wf2, bf2):
    z = jnp.zeros
    wbig = z((6528, DP), _f32)
    wbig = wbig.at[0:21, 0:21].set(W2)
    wbig = wbig.at[21:6165, 21:149].set(W1)
    wbig = wbig.at[6165:6485, 149:469].set(W3)
    bfeat = z((DP,), _f32).at[0:21].set(b2).at[21:149].set(b1)
    bfeat = bfeat.at[149:469].set(b3).reshape(1, DP)

    wc1p = z((DP, DP), _f32).at[:469, :469].set(Wc1)
    wa1p = z((DP, DP), _f32).at[:469, :469].set(Wa1)
    wc2p = z((DP, D2P), _f32).at[:469, :938].set(Wc2)
    wa2p = z((DP, D2P), _f32).at[:469, :938].set(Wa2)
    wc3p = z((D4P, D4P), _f32)
    wc3p = wc3p.at[0:938, 0:1876].set(Wc3[0:938])
    wc3p = wc3p.at[1024:1962, 0:1876].set(Wc3[938:1876])

    bc1c = z((DP,), _f32).at[:469].set(bc1).reshape(4, 1, 128)
    ba1c = z((DP,), _f32).at[:469].set(ba1).reshape(4, 1, 128)
    bc2c = z((D2P,), _f32).at[:938].set(bc2).reshape(8, 1, 128)
    ba2c = z((D2P,), _f32).at[:938].set(ba2).reshape(8, 1, 128)
    bc3c = z((D4P,), _f32).at[:1876].set(bc3).reshape(16, 128)

    wf1p = z((D4P, 1024), _f32).at[:1876].set(Wf1)
    wf2p = z((1024, 512), _f32).at[:, :486].set(Wf2)
    bf2p = z((512,), _f32).at[:486].set(bf2).reshape(1, 512)
    bf1r = bf1.reshape(1, 1024)
    gam = gamma.reshape(1, 1024)
    bet = beta.reshape(1, 1024)

    epad = jnp.broadcast_to(
        (N + jnp.arange(NBR * 128 - EPT, dtype=_i32)) << 16,
        (NSUB, NBR * 128 - EPT))

    def _pe(e):
        packed = e[0] | (e[1] << 16)
        return jnp.concatenate(
            [packed.reshape(NSUB, EPT), epad], axis=1).reshape(NSUB, NBR, 128)

    ed1 = _pe(prot_edge_index)
    ed2 = _pe(edge_index_replace)
    dsts64 = jnp.pad(jnp.concatenate(
        [prot_edge_index[1], edge_index_replace[1]]).reshape(
            2 * NCORE * NSUB, EPW), ((0, 0), (0, 120)))

    degparts = _degree_call(dsts64).reshape(4, 80, 128)
    dv = _dinv_call(degparts)
    dinv1 = dv[0].reshape(NDEG)[:N].reshape(N, 1)
    dinv2 = dv[1].reshape(NDEG)[:N].reshape(N, 1)

    feat = _encoder_call(prot_x, wbig, bfeat)

    h1 = _producer_call(feat, wc1p, dinv1, 4)
    h3 = _producer_call(feat, wa1p, dinv2, 4)
    a1 = _msgpass_call(h1.reshape(4 * N, 128), ed1, 4).reshape(4, NP, 128)
    a3 = _msgpass_call(h3.reshape(4 * N, 128), ed2, 4).reshape(4, NP, 128)

    dv1s4 = jnp.broadcast_to(dinv1[None], (4, N, 1))
    dv2s4 = jnp.broadcast_to(dinv2[None], (4, N, 1))
    h2 = _conprod_call(a1, h1, dv1s4, bc1c, wc2p, dinv1, 8)
    h4 = _conprod_call(a3, h3, dv2s4, ba1c, wa2p, dinv2, 8)
    a2 = _msgpass_call(h2.reshape(8 * N, 128), ed1, 8).reshape(8, NP, 128)
    a4 = _msgpass_call(h4.reshape(8 * N, 128), ed2, 8).reshape(8, NP, 128)

    a24 = jnp.concatenate([a2, a4], axis=0)
    h24 = jnp.concatenate([h2, h4], axis=0)
    dv24 = jnp.concatenate([jnp.broadcast_to(dinv1[None], (8, N, 1)),
                            jnp.broadcast_to(dinv2[None], (8, N, 1))], axis=0)
    b24 = jnp.concatenate([bc2c, ba2c], axis=0)
    h5 = _conprod_call(a24, h24, dv24, b24, wc3p, dinv1, 16)
    a5 = _msgpass_call(h5.reshape(16 * N, 128), ed1, 16).reshape(
        16, NP, 128)

    out = _poolhead_call(a5, h5, dinv1, bc3c, prot_batch.reshape(MB, 1, BM),
                         wf1p, bf1r, gam, bet, wf2p, bf2p)
    return out[:, :486]
